# baseline (speedup 1.0000x reference)
import jax
import jax.numpy as jnp
from jax.experimental import pallas as pl

_B = 16384
_D = 64
_MM_BLK = 1024


def _mm_body(ux_ref, ix_ref, w1_ref, w2_ref, b_ref, o_ref):
    acc = jnp.dot(ux_ref[...], w1_ref[...], preferred_element_type=jnp.float32)
    acc = acc + jnp.dot(ix_ref[...], w2_ref[...], preferred_element_type=jnp.float32)
    o_ref[...] = acc + b_ref[...]


def _tc_matmul(ux, ix, w1t, w2t, b2):
    return pl.pallas_call(
        _mm_body,
        grid=(_B // _MM_BLK,),
        in_specs=[
            pl.BlockSpec((_MM_BLK, _D), lambda i: (i, 0)),
            pl.BlockSpec((_MM_BLK, _D), lambda i: (i, 0)),
            pl.BlockSpec((_D, _D), lambda i: (0, 0)),
            pl.BlockSpec((_D, _D), lambda i: (0, 0)),
            pl.BlockSpec((1, _D), lambda i: (0, 0)),
        ],
        out_specs=pl.BlockSpec((_MM_BLK, _D), lambda i: (i, 0)),
        out_shape=jax.ShapeDtypeStruct((_B, _D), jnp.float32),
    )(ux, ix, w1t, w2t, b2)


def kernel(x, user_table, item_table, W, b):
    ux = jnp.take(user_table, x[:, 0], axis=0, mode="clip")
    ix = jnp.take(item_table, x[:, 1], axis=0, mode="clip")
    w1t = W[:, :_D].T
    w2t = W[:, _D:].T
    return _tc_matmul(ux, ix, w1t, w2t, b.reshape(1, _D))
